# Initial kernel scaffold; baseline (speedup 1.0000x reference)
import jax
import jax.numpy as jnp
from jax.experimental import pallas as pl


def kernel(x, params):
    raise NotImplementedError("write your pallas kernel here")


# speedup vs baseline: 1.6483x; 1.6483x over previous
import jax
import jax.numpy as jnp
from jax.experimental import pallas as pl

B = 16
T = 512
N = 128
P = 96
E = 8


def _top2_gate(xn, gate_w):
    feat = jnp.mean(xn, axis=1)[None, :]
    logits = jnp.dot(feat, gate_w,
                     preferred_element_type=jnp.float32,
                     precision=jax.lax.Precision.HIGHEST)
    eids = jax.lax.broadcasted_iota(jnp.int32, (1, E), 1)
    v0 = jnp.max(logits)
    i0 = jnp.argmax(logits, axis=1)[0]
    masked = jnp.where(eids == i0, -jnp.inf, logits)
    v1 = jnp.max(masked)
    i1 = jnp.argmax(masked, axis=1)[0]
    e1 = jnp.exp(v1 - v0)
    g0 = 1.0 / (1.0 + e1)
    g1 = e1 / (1.0 + e1)
    gates_row = (jnp.where(eids == i0, g0, 0.0)
                 + jnp.where(eids == i1, g1, 0.0))
    return i0, i1, g0, g1, gates_row


def _expert_mix(xn, exp_w_ref, exp_b_ref, i0, i1, g0, g1):
    w0 = exp_w_ref[i0]
    w1 = exp_w_ref[i1]
    dn = (((0,), (0,)), ((), ()))
    a0 = jax.lax.dot_general(w0, xn, dn, preferred_element_type=jnp.float32)
    a1 = jax.lax.dot_general(w1, xn, dn, preferred_element_type=jnp.float32)
    bias = g0 * exp_b_ref[i0] + g1 * exp_b_ref[i1]
    return g0 * a0 + g1 * a1 + bias + xn


def _stage1_kernel(x_ref, gate_w_ref, exp_w_ref, exp_b_ref,
                   out_ref, gates_ref):
    xb = x_ref[0]
    m = jnp.mean(xb, axis=0, keepdims=True)
    var = jnp.mean((xb - m) ** 2, axis=0, keepdims=True)
    xn = (xb - m) * jax.lax.rsqrt(var + 1e-5)
    i0, i1, g0, g1, gates_row = _top2_gate(xn, gate_w_ref[...])
    gates_ref[0] = gates_row
    out_ref[0] = _expert_mix(xn, exp_w_ref, exp_b_ref, i0, i1, g0, g1)


def _stage2_kernel(x_ref, gate_w_ref, exp_w_ref, exp_b_ref,
                   p1w_ref, p1b_ref, wm_ref, bm_ref, ws_ref, bs_ref,
                   mean_ref, std_ref, gates_ref):
    xn = x_ref[0]
    i0, i1, g0, g1, gates_row = _top2_gate(xn, gate_w_ref[...])
    gates_ref[0] = gates_row
    out = _expert_mix(xn, exp_w_ref, exp_b_ref, i0, i1, g0, g1)
    h = jnp.tanh(jnp.dot(out.T, p1w_ref[...],
                         preferred_element_type=jnp.float32) + p1b_ref[...])
    mean_bn = jnp.dot(h, wm_ref[...],
                      preferred_element_type=jnp.float32) + bm_ref[...]
    std_bn = jnp.dot(h, ws_ref[...],
                     preferred_element_type=jnp.float32) + bs_ref[...]
    mean_ref[0] = mean_bn.T
    std_ref[0] = jax.nn.softplus(std_bn).T + 1e-6


def _balance_kernel(g0_ref, g1_ref, out_ref):
    def aux(g):
        imp = jnp.sum(g[:, 0, :], axis=0, keepdims=True)
        mu = jnp.mean(imp)
        var = jnp.mean((imp - mu) ** 2)
        return var / (mu * mu + 1e-10)

    out_ref[...] = jnp.reshape(aux(g0_ref[...]) + aux(g1_ref[...]), (1, 1))


def kernel(x, params):
    p = params
    xs = x[..., 0]
    grid = (B,)

    out0, gates0 = pl.pallas_call(
        _stage1_kernel,
        grid=grid,
        in_specs=[
            pl.BlockSpec((1, T, N), lambda b: (b, 0, 0)),
            pl.BlockSpec((T, E), lambda b: (0, 0)),
            pl.BlockSpec((E, T, T), lambda b: (0, 0, 0)),
            pl.BlockSpec((E, T, 1), lambda b: (0, 0, 0)),
        ],
        out_specs=[
            pl.BlockSpec((1, T, N), lambda b: (b, 0, 0)),
            pl.BlockSpec((1, 1, E), lambda b: (b, 0, 0)),
        ],
        out_shape=[
            jax.ShapeDtypeStruct((B, T, N), jnp.float32),
            jax.ShapeDtypeStruct((B, 1, E), jnp.float32),
        ],
    )(xs, p['l0_gate_w'], p['l0_exp_w'], p['l0_exp_b'][:, :, None])

    p2w, p2b = p['proj2_w'], p['proj2_b']
    mean, std, gates1 = pl.pallas_call(
        _stage2_kernel,
        grid=grid,
        in_specs=[
            pl.BlockSpec((1, T, N), lambda b: (b, 0, 0)),
            pl.BlockSpec((T, E), lambda b: (0, 0)),
            pl.BlockSpec((E, T, T), lambda b: (0, 0, 0)),
            pl.BlockSpec((E, T, 1), lambda b: (0, 0, 0)),
            pl.BlockSpec((T, P), lambda b: (0, 0)),
            pl.BlockSpec((1, P), lambda b: (0, 0)),
            pl.BlockSpec((P, P), lambda b: (0, 0)),
            pl.BlockSpec((1, P), lambda b: (0, 0)),
            pl.BlockSpec((P, P), lambda b: (0, 0)),
            pl.BlockSpec((1, P), lambda b: (0, 0)),
        ],
        out_specs=[
            pl.BlockSpec((1, P, N), lambda b: (b, 0, 0)),
            pl.BlockSpec((1, P, N), lambda b: (b, 0, 0)),
            pl.BlockSpec((1, 1, E), lambda b: (b, 0, 0)),
        ],
        out_shape=[
            jax.ShapeDtypeStruct((B, P, N), jnp.float32),
            jax.ShapeDtypeStruct((B, P, N), jnp.float32),
            jax.ShapeDtypeStruct((B, 1, E), jnp.float32),
        ],
    )(out0, p['l1_gate_w'], p['l1_exp_w'], p['l1_exp_b'][:, :, None],
      p['proj1_w'], p['proj1_b'][None, :],
      p2w[:, 0::2], p2b[None, 0::2], p2w[:, 1::2], p2b[None, 1::2])

    balance = pl.pallas_call(
        _balance_kernel,
        out_shape=jax.ShapeDtypeStruct((1, 1), jnp.float32),
    )(gates0, gates1)[0, 0]

    return (mean, balance, std)


# speedup vs baseline: 2.0955x; 1.2713x over previous
import jax
import jax.numpy as jnp
from jax.experimental import pallas as pl

B = 16
T = 512
N = 128
P = 96
E = 8


def _top2_gate(y, gate_w):
    feat = jnp.mean(y, axis=0, keepdims=True)
    logits = jnp.dot(feat, gate_w,
                     preferred_element_type=jnp.float32,
                     precision=jax.lax.Precision.HIGHEST)
    eids = jax.lax.broadcasted_iota(jnp.int32, (1, E), 1)
    v0 = jnp.max(logits)
    i0 = jnp.argmax(logits, axis=1)[0]
    masked = jnp.where(eids == i0, -jnp.inf, logits)
    v1 = jnp.max(masked)
    i1 = jnp.argmax(masked, axis=1)[0]
    e1 = jnp.exp(v1 - v0)
    g0 = 1.0 / (1.0 + e1)
    g1 = e1 / (1.0 + e1)
    gates_row = (jnp.where(eids == i0, g0, 0.0)
                 + jnp.where(eids == i1, g1, 0.0))
    return i0, i1, g0, g1, gates_row


def _moe_layer(y, gate_w, exp_w_ref, exp_b_ref):
    i0, i1, g0, g1, gates_row = _top2_gate(y, gate_w)
    w0 = exp_w_ref[i0]
    w1 = exp_w_ref[i1]
    a0 = jnp.dot(y, w0, preferred_element_type=jnp.float32)
    a1 = jnp.dot(y, w1, preferred_element_type=jnp.float32)
    bias = g0 * exp_b_ref[i0] + g1 * exp_b_ref[i1]
    return g0 * a0 + g1 * a1 + bias + y, gates_row


def _fused_kernel(y_ref, g0w_ref, w0_ref, b0_ref, g1w_ref, w1_ref, b1_ref,
                  p1w_ref, p1b_ref, wm_ref, bm_ref, ws_ref, bs_ref,
                  mean_ref, std_ref, gates0_ref, gates1_ref):
    yb = y_ref[0]
    m = jnp.mean(yb, axis=1, keepdims=True)
    var = jnp.mean((yb - m) ** 2, axis=1, keepdims=True)
    y = (yb - m) * jax.lax.rsqrt(var + 1e-5)
    y, gates0 = _moe_layer(y, g0w_ref[...], w0_ref, b0_ref)
    gates0_ref[0] = gates0
    y, gates1 = _moe_layer(y, g1w_ref[...], w1_ref, b1_ref)
    gates1_ref[0] = gates1
    h = jnp.tanh(jnp.dot(y, p1w_ref[...],
                         preferred_element_type=jnp.float32) + p1b_ref[...])
    mean_bn = jnp.dot(h, wm_ref[...],
                      preferred_element_type=jnp.float32) + bm_ref[...]
    std_bn = jnp.dot(h, ws_ref[...],
                     preferred_element_type=jnp.float32) + bs_ref[...]
    mean_ref[0] = mean_bn.T
    std_ref[0] = jax.nn.softplus(std_bn).T + 1e-6


def _balance_kernel(g0_ref, g1_ref, out_ref):
    def aux(g):
        imp = jnp.sum(g[:, 0, :], axis=0, keepdims=True)
        mu = jnp.mean(imp)
        var = jnp.mean((imp - mu) ** 2)
        return var / (mu * mu + 1e-10)

    out_ref[...] = jnp.reshape(aux(g0_ref[...]) + aux(g1_ref[...]), (1, 1))


def kernel(x, params):
    p = params
    ys = jnp.transpose(x[..., 0], (0, 2, 1))
    p2w, p2b = p['proj2_w'], p['proj2_b']

    _const = lambda *dims: pl.BlockSpec(dims, lambda b: (0,) * len(dims))
    mean, std, gates0, gates1 = pl.pallas_call(
        _fused_kernel,
        grid=(B,),
        in_specs=[
            pl.BlockSpec((1, N, T), lambda b: (b, 0, 0)),
            _const(T, E), _const(E, T, T), _const(E, 1, T),
            _const(T, E), _const(E, T, T), _const(E, 1, T),
            _const(T, P), _const(1, P),
            _const(P, P), _const(1, P), _const(P, P), _const(1, P),
        ],
        out_specs=[
            pl.BlockSpec((1, P, N), lambda b: (b, 0, 0)),
            pl.BlockSpec((1, P, N), lambda b: (b, 0, 0)),
            pl.BlockSpec((1, 1, E), lambda b: (b, 0, 0)),
            pl.BlockSpec((1, 1, E), lambda b: (b, 0, 0)),
        ],
        out_shape=[
            jax.ShapeDtypeStruct((B, P, N), jnp.float32),
            jax.ShapeDtypeStruct((B, P, N), jnp.float32),
            jax.ShapeDtypeStruct((B, 1, E), jnp.float32),
            jax.ShapeDtypeStruct((B, 1, E), jnp.float32),
        ],
    )(ys,
      p['l0_gate_w'], p['l0_exp_w'], p['l0_exp_b'][:, None, :],
      p['l1_gate_w'], p['l1_exp_w'], p['l1_exp_b'][:, None, :],
      p['proj1_w'], p['proj1_b'][None, :],
      p2w[:, 0::2], p2b[None, 0::2], p2w[:, 1::2], p2b[None, 1::2])

    balance = pl.pallas_call(
        _balance_kernel,
        out_shape=jax.ShapeDtypeStruct((1, 1), jnp.float32),
    )(gates0, gates1)[0, 0]

    return (mean, balance, std)
